# baseline (speedup 1.0000x reference)
import functools

import jax
import jax.numpy as jnp
from jax.experimental import pallas as pl
from jax.experimental.pallas import tpu as pltpu

_K = 20


def _oem_merge(lo, n, r):
    step = r * 2
    if step < n:
        yield from _oem_merge(lo, n, step)
        yield from _oem_merge(lo + r, n, step)
        for i in range(lo + r, lo + n - r, step):
            yield (i, i + r)
    else:
        yield (lo, lo + r)


def _oem_sort(lo, n):
    if n > 1:
        m = n // 2
        yield from _oem_sort(lo, m)
        yield from _oem_sort(lo + m, m)
        yield from _oem_merge(lo, n, 1)


def _topk_threshold(dsub, k):
    n = dsub.shape[1]
    nchunk = n // 128
    ch = [dsub[:, c * 128:(c + 1) * 128] for c in range(nchunk)]
    for i, l in _oem_sort(0, nchunk):
        hi = jnp.maximum(ch[i], ch[l])
        ch[l] = jnp.minimum(ch[i], ch[l])
        ch[i] = hi
    neg_inf = jnp.full_like(ch[0], -jnp.inf)
    s0 = None
    t = None
    for it in range(k):
        m = jnp.max(ch[0], axis=1, keepdims=True)
        if it == 0:
            s0 = m
        if it == k - 1:
            t = m
        else:
            mask = ch[0] == m
            depth = min(nchunk - 1, k - 2 - it)
            for j in range(depth + 1):
                nxt = ch[j + 1] if j + 1 < nchunk else neg_inf
                ch[j] = jnp.where(mask, nxt, ch[j])
    return t, s0


def _attn_conv_kernel(xt_ref, w_ref, y_ref, coln_ref, *, k, tq, rs):
    q = pl.program_id(1)
    xt = xt_ref[0]
    tile = xt_ref[0, pl.ds(q * tq, tq), :]

    @pl.when(q == 0)
    def _():
        coln_ref[...] = jnp.sum(xt * xt, axis=1)[None, :]

    coln = coln_ref[...]
    d = 2.0 * jax.lax.dot_general(
        tile, xt, (((1,), (1,)), ((), ())),
        preferred_element_type=jnp.float32)
    d = d - coln

    tparts = []
    s0parts = []
    for s in range(tq // rs):
        ts, s0s = _topk_threshold(d[s * rs:(s + 1) * rs], k)
        tparts.append(ts)
        s0parts.append(s0s)
    t = jnp.concatenate(tparts, axis=0)
    s0 = jnp.concatenate(s0parts, axis=0)

    w = jnp.where(d >= t, jnp.exp(d - s0), 0.0)
    den = jnp.sum(w, axis=1, keepdims=True)
    agg = jax.lax.dot_general(
        w, xt, (((1,), (0,)), ((), ())),
        preferred_element_type=jnp.float32)
    wnn = agg / den - tile
    att = jnp.concatenate([tile, wnn], axis=1)
    y = jax.lax.dot_general(
        att, w_ref[...], (((1,), (1,)), ((), ())),
        preferred_element_type=jnp.float32)
    y_ref[0] = y


def _bn_lrelu_kernel(y_ref, g_ref, b_ref, x_ref):
    y = y_ref[...]
    c = y.shape[-1]
    y2 = y.reshape(-1, c)
    m = jnp.mean(y2, axis=0, keepdims=True)
    v = jnp.mean((y2 - m) ** 2, axis=0, keepdims=True)
    xn = (y2 - m) / jnp.sqrt(v + 1e-5)
    xn = xn * g_ref[...] + b_ref[...]
    x = jnp.where(xn > 0, xn, 0.2 * xn)
    x_ref[...] = x.reshape(y.shape)


def _round(xt, w, g, b, tq, rs):
    bsz, n, c = xt.shape
    cout = w.shape[0]
    y = pl.pallas_call(
        functools.partial(_attn_conv_kernel, k=_K, tq=tq, rs=rs),
        grid=(bsz, n // tq),
        in_specs=[
            pl.BlockSpec((1, n, c), lambda bi, qi: (bi, 0, 0)),
            pl.BlockSpec((cout, 2 * c), lambda bi, qi: (0, 0)),
        ],
        out_specs=pl.BlockSpec((1, tq, cout), lambda bi, qi: (bi, qi, 0)),
        out_shape=jax.ShapeDtypeStruct((bsz, n, cout), jnp.float32),
        scratch_shapes=[pltpu.VMEM((1, n), jnp.float32)],
    )(xt, w)
    x = pl.pallas_call(
        _bn_lrelu_kernel,
        in_specs=[
            pl.BlockSpec((bsz, n, cout), lambda: (0, 0, 0)),
            pl.BlockSpec((1, cout), lambda: (0, 0)),
            pl.BlockSpec((1, cout), lambda: (0, 0)),
        ],
        out_specs=pl.BlockSpec((bsz, n, cout), lambda: (0, 0, 0)),
        out_shape=jax.ShapeDtypeStruct((bsz, n, cout), jnp.float32),
    )(y, g.reshape(1, cout), b.reshape(1, cout))
    return x


def kernel(x, W1, g1, b1, W2, g2, b2, W3, g3, b3):
    xt = jnp.transpose(x, (0, 2, 1))
    x1 = _round(xt, W1, g1, b1, 256, 64)
    x2 = _round(x1, W2, g2, b2, 256, 64)
    x3 = _round(x2, W3, g3, b3, 256, 64)
    return (jnp.transpose(x1, (0, 2, 1)),
            jnp.transpose(x2, (0, 2, 1)),
            jnp.transpose(x3, (0, 2, 1)))


# speedup vs baseline: 1.0613x; 1.0613x over previous
import functools

import jax
import jax.numpy as jnp
from jax.experimental import pallas as pl
from jax.experimental.pallas import tpu as pltpu

_K = 20


def _oem_merge(lo, n, r):
    step = r * 2
    if step < n:
        yield from _oem_merge(lo, n, step)
        yield from _oem_merge(lo + r, n, step)
        for i in range(lo + r, lo + n - r, step):
            yield (i, i + r)
    else:
        yield (lo, lo + r)


def _oem_sort(lo, n):
    if n > 1:
        m = n // 2
        yield from _oem_sort(lo, m)
        yield from _oem_sort(lo + m, m)
        yield from _oem_merge(lo, n, 1)


def _topk_threshold(dsub, k):
    n = dsub.shape[1]
    nchunk = n // 128
    ch = [dsub[:, c * 128:(c + 1) * 128] for c in range(nchunk)]
    for i, l in _oem_sort(0, nchunk):
        hi = jnp.maximum(ch[i], ch[l])
        ch[l] = jnp.minimum(ch[i], ch[l])
        ch[i] = hi
    neg_inf = jnp.full_like(ch[0], -jnp.inf)
    s0 = None
    t = None
    for it in range(k):
        m = jnp.max(ch[0], axis=1, keepdims=True)
        if it == 0:
            s0 = m
        if it == k - 1:
            t = m
        else:
            mask = ch[0] == m
            depth = min(nchunk - 1, k - 2 - it)
            for j in range(depth + 1):
                nxt = ch[j + 1] if j + 1 < nchunk else neg_inf
                ch[j] = jnp.where(mask, nxt, ch[j])
    return t, s0


def _attn_conv_kernel(xt_ref, w_ref, y_ref, coln_ref, *, k, tq, rs):
    q = pl.program_id(1)
    xt = xt_ref[0]
    tile = xt_ref[0, pl.ds(q * tq, tq), :]

    @pl.when(q == 0)
    def _():
        coln_ref[...] = jnp.sum(xt * xt, axis=1)[None, :]

    coln = coln_ref[...]
    d = 2.0 * jax.lax.dot_general(
        tile, xt, (((1,), (1,)), ((), ())),
        preferred_element_type=jnp.float32)
    d = d - coln

    tparts = []
    s0parts = []
    for s in range(tq // rs):
        ts, s0s = _topk_threshold(d[s * rs:(s + 1) * rs], k)
        tparts.append(ts)
        s0parts.append(s0s)
    t = jnp.concatenate(tparts, axis=0)
    s0 = jnp.concatenate(s0parts, axis=0)

    w = jnp.where(d >= t, jnp.exp(d - s0), 0.0)
    den = jnp.sum(w, axis=1, keepdims=True)
    agg = jax.lax.dot_general(
        w, xt, (((1,), (0,)), ((), ())),
        preferred_element_type=jnp.float32)
    wnn = agg / den - tile
    att = jnp.concatenate([tile, wnn], axis=1)
    y = jax.lax.dot_general(
        att, w_ref[...], (((1,), (1,)), ((), ())),
        preferred_element_type=jnp.float32)
    y_ref[0] = y


def _bn_lrelu_kernel(y_ref, g_ref, b_ref, x_ref):
    y = y_ref[...]
    c = y.shape[-1]
    y2 = y.reshape(-1, c)
    m = jnp.mean(y2, axis=0, keepdims=True)
    v = jnp.mean((y2 - m) ** 2, axis=0, keepdims=True)
    xn = (y2 - m) / jnp.sqrt(v + 1e-5)
    xn = xn * g_ref[...] + b_ref[...]
    x = jnp.where(xn > 0, xn, 0.2 * xn)
    x_ref[...] = x.reshape(y.shape)


def _round(xt, w, g, b, tq, rs):
    bsz, n, c = xt.shape
    cout = w.shape[0]
    y = pl.pallas_call(
        functools.partial(_attn_conv_kernel, k=_K, tq=tq, rs=rs),
        grid=(bsz, n // tq),
        in_specs=[
            pl.BlockSpec((1, n, c), lambda bi, qi: (bi, 0, 0)),
            pl.BlockSpec((cout, 2 * c), lambda bi, qi: (0, 0)),
        ],
        out_specs=pl.BlockSpec((1, tq, cout), lambda bi, qi: (bi, qi, 0)),
        out_shape=jax.ShapeDtypeStruct((bsz, n, cout), jnp.float32),
        scratch_shapes=[pltpu.VMEM((1, n), jnp.float32)],
    )(xt, w)
    x = pl.pallas_call(
        _bn_lrelu_kernel,
        in_specs=[
            pl.BlockSpec((bsz, n, cout), lambda: (0, 0, 0)),
            pl.BlockSpec((1, cout), lambda: (0, 0)),
            pl.BlockSpec((1, cout), lambda: (0, 0)),
        ],
        out_specs=pl.BlockSpec((bsz, n, cout), lambda: (0, 0, 0)),
        out_shape=jax.ShapeDtypeStruct((bsz, n, cout), jnp.float32),
    )(y, g.reshape(1, cout), b.reshape(1, cout))
    return x


def kernel(x, W1, g1, b1, W2, g2, b2, W3, g3, b3):
    xt = jnp.transpose(x, (0, 2, 1))
    x1 = _round(xt, W1, g1, b1, 256, 256)
    x2 = _round(x1, W2, g2, b2, 256, 256)
    x3 = _round(x2, W3, g3, b3, 256, 256)
    return (jnp.transpose(x1, (0, 2, 1)),
            jnp.transpose(x2, (0, 2, 1)),
            jnp.transpose(x3, (0, 2, 1)))


# speedup vs baseline: 1.0850x; 1.0223x over previous
import functools

import jax
import jax.numpy as jnp
from jax.experimental import pallas as pl
from jax.experimental.pallas import tpu as pltpu

_K = 20


def _oem_merge(lo, n, r):
    step = r * 2
    if step < n:
        yield from _oem_merge(lo, n, step)
        yield from _oem_merge(lo + r, n, step)
        for i in range(lo + r, lo + n - r, step):
            yield (i, i + r)
    else:
        yield (lo, lo + r)


def _oem_sort(lo, n):
    if n > 1:
        m = n // 2
        yield from _oem_sort(lo, m)
        yield from _oem_sort(lo + m, m)
        yield from _oem_merge(lo, n, 1)


def _topk_threshold(dsub, k):
    n = dsub.shape[1]
    nchunk = n // 128
    ch = [dsub[:, c * 128:(c + 1) * 128] for c in range(nchunk)]
    for i, l in _oem_sort(0, nchunk):
        hi = jnp.maximum(ch[i], ch[l])
        ch[l] = jnp.minimum(ch[i], ch[l])
        ch[i] = hi
    neg_inf = jnp.full_like(ch[0], -jnp.inf)
    s0 = None
    t = None
    for it in range(k):
        m = jnp.max(ch[0], axis=1, keepdims=True)
        if it == 0:
            s0 = m
        if it == k - 1:
            t = m
        else:
            mask = ch[0] == m
            depth = min(nchunk - 1, k - 2 - it)
            for j in range(depth + 1):
                nxt = ch[j + 1] if j + 1 < nchunk else neg_inf
                ch[j] = jnp.where(mask, nxt, ch[j])
    return t, s0


def _attn_conv_kernel(xt_ref, w_ref, y_ref, coln_ref, *, k, tq, rs):
    q = pl.program_id(1)
    xt = xt_ref[0]
    tile = xt_ref[0, pl.ds(q * tq, tq), :]

    @pl.when(q == 0)
    def _():
        coln_ref[...] = jnp.sum(xt * xt, axis=1)[None, :]

    coln = coln_ref[...]
    d = 2.0 * jax.lax.dot_general(
        tile, xt, (((1,), (1,)), ((), ())),
        preferred_element_type=jnp.float32)
    d = d - coln

    tparts = []
    s0parts = []
    for s in range(tq // rs):
        ts, s0s = _topk_threshold(d[s * rs:(s + 1) * rs], k)
        tparts.append(ts)
        s0parts.append(s0s)
    t = jnp.concatenate(tparts, axis=0)
    s0 = jnp.concatenate(s0parts, axis=0)

    w = jnp.where(d >= t, jnp.exp(d - s0), 0.0)
    den = jnp.sum(w, axis=1, keepdims=True)
    agg = jax.lax.dot_general(
        w, xt, (((1,), (0,)), ((), ())),
        preferred_element_type=jnp.float32)
    wnn = agg / den - tile
    att = jnp.concatenate([tile, wnn], axis=1)
    y = jax.lax.dot_general(
        att, w_ref[...], (((1,), (1,)), ((), ())),
        preferred_element_type=jnp.float32)
    y_ref[0] = y


def _bn_lrelu_kernel(y_ref, g_ref, b_ref, x_ref):
    y = y_ref[...]
    c = y.shape[-1]
    y2 = y.reshape(-1, c)
    m = jnp.mean(y2, axis=0, keepdims=True)
    v = jnp.mean((y2 - m) ** 2, axis=0, keepdims=True)
    xn = (y2 - m) / jnp.sqrt(v + 1e-5)
    xn = xn * g_ref[...] + b_ref[...]
    x = jnp.where(xn > 0, xn, 0.2 * xn)
    x_ref[...] = x.reshape(y.shape)


def _round(xt, w, g, b, tq, rs):
    bsz, n, c = xt.shape
    cout = w.shape[0]
    y = pl.pallas_call(
        functools.partial(_attn_conv_kernel, k=_K, tq=tq, rs=rs),
        grid=(bsz, n // tq),
        in_specs=[
            pl.BlockSpec((1, n, c), lambda bi, qi: (bi, 0, 0)),
            pl.BlockSpec((cout, 2 * c), lambda bi, qi: (0, 0)),
        ],
        out_specs=pl.BlockSpec((1, tq, cout), lambda bi, qi: (bi, qi, 0)),
        out_shape=jax.ShapeDtypeStruct((bsz, n, cout), jnp.float32),
        scratch_shapes=[pltpu.VMEM((1, n), jnp.float32)],
    )(xt, w)
    x = pl.pallas_call(
        _bn_lrelu_kernel,
        in_specs=[
            pl.BlockSpec((bsz, n, cout), lambda: (0, 0, 0)),
            pl.BlockSpec((1, cout), lambda: (0, 0)),
            pl.BlockSpec((1, cout), lambda: (0, 0)),
        ],
        out_specs=pl.BlockSpec((bsz, n, cout), lambda: (0, 0, 0)),
        out_shape=jax.ShapeDtypeStruct((bsz, n, cout), jnp.float32),
    )(y, g.reshape(1, cout), b.reshape(1, cout))
    return x


def kernel(x, W1, g1, b1, W2, g2, b2, W3, g3, b3):
    xt = jnp.transpose(x, (0, 2, 1))
    x1 = _round(xt, W1, g1, b1, 512, 512)
    x2 = _round(x1, W2, g2, b2, 512, 512)
    x3 = _round(x2, W3, g3, b3, 512, 512)
    return (jnp.transpose(x1, (0, 2, 1)),
            jnp.transpose(x2, (0, 2, 1)),
            jnp.transpose(x3, (0, 2, 1)))


# speedup vs baseline: 1.1310x; 1.0424x over previous
import functools

import jax
import jax.numpy as jnp
from jax.experimental import pallas as pl
from jax.experimental.pallas import tpu as pltpu

_K = 20


def _oem_merge(lo, n, r):
    step = r * 2
    if step < n:
        yield from _oem_merge(lo, n, step)
        yield from _oem_merge(lo + r, n, step)
        for i in range(lo + r, lo + n - r, step):
            yield (i, i + r)
    else:
        yield (lo, lo + r)


def _oem_sort(lo, n):
    if n > 1:
        m = n // 2
        yield from _oem_sort(lo, m)
        yield from _oem_sort(lo + m, m)
        yield from _oem_merge(lo, n, 1)


def _topk_threshold(dsub, k):
    n = dsub.shape[1]
    nchunk = n // 128
    ch = [dsub[:, c * 128:(c + 1) * 128] for c in range(nchunk)]
    for i, l in _oem_sort(0, nchunk):
        hi = jnp.maximum(ch[i], ch[l])
        ch[l] = jnp.minimum(ch[i], ch[l])
        ch[i] = hi
    neg_inf = jnp.full_like(ch[0], -jnp.inf)
    s0 = None
    t = None
    for it in range(k):
        m = jnp.max(ch[0], axis=1, keepdims=True)
        if it == 0:
            s0 = m
        if it == k - 1:
            t = m
        else:
            mask = ch[0] == m
            depth = min(nchunk - 1, k - 2 - it)
            for j in range(depth + 1):
                nxt = ch[j + 1] if j + 1 < nchunk else neg_inf
                ch[j] = jnp.where(mask, nxt, ch[j])
    return t, s0


def _attn_conv_kernel(xt_ref, w_ref, y_ref, aug_ref, *, k, tq, rs):
    q = pl.program_id(1)
    xt = xt_ref[0]
    tile = xt_ref[0, pl.ds(q * tq, tq), :]

    @pl.when(q == 0)
    def _():
        coln = jnp.sum(xt * xt, axis=1, keepdims=True)
        aug_ref[...] = jnp.concatenate([2.0 * xt, -coln], axis=1)

    ones = jnp.ones((tq, 1), jnp.float32)
    tile_aug = jnp.concatenate([tile, ones], axis=1)
    d = jax.lax.dot_general(
        tile_aug, aug_ref[...], (((1,), (1,)), ((), ())),
        preferred_element_type=jnp.float32)

    tparts = []
    s0parts = []
    for s in range(tq // rs):
        ts, s0s = _topk_threshold(d[s * rs:(s + 1) * rs], k)
        tparts.append(ts)
        s0parts.append(s0s)
    t = jnp.concatenate(tparts, axis=0)
    s0 = jnp.concatenate(s0parts, axis=0)

    w = jnp.where(d >= t, jnp.exp(d - s0), 0.0)
    den = jnp.sum(w, axis=1, keepdims=True)
    agg = jax.lax.dot_general(
        w, xt, (((1,), (0,)), ((), ())),
        preferred_element_type=jnp.float32)
    wnn = agg / den - tile
    att = jnp.concatenate([tile, wnn], axis=1)
    y = jax.lax.dot_general(
        att, w_ref[...], (((1,), (1,)), ((), ())),
        preferred_element_type=jnp.float32)
    y_ref[0] = y


def _bn_lrelu_kernel(y_ref, g_ref, b_ref, x_ref):
    y = y_ref[...]
    c = y.shape[-1]
    y2 = y.reshape(-1, c)
    m = jnp.mean(y2, axis=0, keepdims=True)
    v = jnp.mean((y2 - m) ** 2, axis=0, keepdims=True)
    xn = (y2 - m) / jnp.sqrt(v + 1e-5)
    xn = xn * g_ref[...] + b_ref[...]
    x = jnp.where(xn > 0, xn, 0.2 * xn)
    x_ref[...] = x.reshape(y.shape)


def _round(xt, w, g, b, tq, rs):
    bsz, n, c = xt.shape
    cout = w.shape[0]
    y = pl.pallas_call(
        functools.partial(_attn_conv_kernel, k=_K, tq=tq, rs=rs),
        grid=(bsz, n // tq),
        in_specs=[
            pl.BlockSpec((1, n, c), lambda bi, qi: (bi, 0, 0)),
            pl.BlockSpec((cout, 2 * c), lambda bi, qi: (0, 0)),
        ],
        out_specs=pl.BlockSpec((1, tq, cout), lambda bi, qi: (bi, qi, 0)),
        out_shape=jax.ShapeDtypeStruct((bsz, n, cout), jnp.float32),
        scratch_shapes=[pltpu.VMEM((n, c + 1), jnp.float32)],
    )(xt, w)
    x = pl.pallas_call(
        _bn_lrelu_kernel,
        in_specs=[
            pl.BlockSpec((bsz, n, cout), lambda: (0, 0, 0)),
            pl.BlockSpec((1, cout), lambda: (0, 0)),
            pl.BlockSpec((1, cout), lambda: (0, 0)),
        ],
        out_specs=pl.BlockSpec((bsz, n, cout), lambda: (0, 0, 0)),
        out_shape=jax.ShapeDtypeStruct((bsz, n, cout), jnp.float32),
    )(y, g.reshape(1, cout), b.reshape(1, cout))
    return x


def kernel(x, W1, g1, b1, W2, g2, b2, W3, g3, b3):
    xt = jnp.transpose(x, (0, 2, 1))
    x1 = _round(xt, W1, g1, b1, 512, 512)
    x2 = _round(x1, W2, g2, b2, 512, 512)
    x3 = _round(x2, W3, g3, b3, 512, 512)
    return (jnp.transpose(x1, (0, 2, 1)),
            jnp.transpose(x2, (0, 2, 1)),
            jnp.transpose(x3, (0, 2, 1)))
